# Initial kernel scaffold; baseline (speedup 1.0000x reference)
import jax
import jax.numpy as jnp
from jax import lax
from jax.experimental import pallas as pl
from jax.experimental.pallas import tpu as pltpu
from jax.experimental.pallas import tpu_sc as plsc

N_ROWS = 320000
D = 128
NSEG = 1024
L = 16
NF = D // L
C = 256
NCHUNK = N_ROWS // C
NWORK = 32
SEG_PER_W = NSEG // NWORK
GROUPS = C // L


def _row_vecs(xbuf, row):
  return [xbuf[row, pl.ds(f * L, L)] for f in range(NF)]


def _body(x_hbm, batch_hbm, out_hbm, xbuf, idbuf, outbuf, probe):
  wid = lax.axis_index("s") * 2 + lax.axis_index("c")
  base_seg = wid * SEG_PER_W
  minus_inf = jnp.full((L,), -jnp.inf, jnp.float32)
  iota = lax.iota(jnp.int32, L)

  def init_body(s, carry):
    for f in range(NF):
      outbuf[s, pl.ds(f * L, L)] = minus_inf
    return carry

  lax.fori_loop(0, SEG_PER_W, init_body, 0)

  def count_below(target):
    pos = jnp.int32(0)
    step = 1
    while step * 2 <= NCHUNK:
      step *= 2
    while step >= 1:
      nxt = pos + step
      idx = jnp.minimum(nxt, NCHUNK) - 1
      pltpu.sync_copy(batch_hbm.at[pl.ds(idx * C, L)], probe)
      val = jnp.min(probe[...])
      ok = jnp.logical_and(nxt <= NCHUNK, val < target)
      pos = jnp.where(ok, nxt, pos)
      step //= 2
    return pos

  n_lo = count_below(base_seg)
  k_start = jnp.maximum(n_lo - 1, 0)
  k_end = count_below(base_seg + SEG_PER_W)

  def flush(cur, accs):
    rel = cur - base_seg
    guard = jnp.logical_and(rel >= 0, rel < SEG_PER_W)

    @pl.when(guard)
    def _():
      for f in range(NF):
        outbuf[rel, pl.ds(f * L, L)] = accs[f]

  def group_body(g, carry):
    cur, accs = carry
    ids16 = idbuf[pl.ds(g * L, L)]
    lo = jnp.min(ids16)
    hi = jnp.max(ids16)
    fast = jnp.logical_and(lo == cur, hi == cur)

    def fast_fn(cur, accs):
      for r in range(L):
        row = _row_vecs(xbuf, g * L + r)
        accs = [jnp.maximum(accs[f], row[f]) for f in range(NF)]
      return cur, accs

    def slow_fn(cur, accs):
      for r in range(L):
        id_r = jnp.max(jnp.where(iota == r, ids16, jnp.int32(-2147483647)))
        row = _row_vecs(xbuf, g * L + r)
        changed = id_r != cur

        @pl.when(changed)
        def _():
          flush(cur, accs)

        accs = [
            jnp.where(changed, row[f], jnp.maximum(accs[f], row[f]))
            for f in range(NF)
        ]
        cur = id_r
      return cur, accs

    return lax.cond(fast, fast_fn, slow_fn, cur, accs)

  def chunk_body(k, carry):
    pltpu.sync_copy(x_hbm.at[pl.ds(k * C, C)], xbuf)
    pltpu.sync_copy(batch_hbm.at[pl.ds(k * C, C)], idbuf)
    return lax.fori_loop(0, GROUPS, group_body, carry)

  accs0 = [jnp.full((L,), -jnp.inf, jnp.float32) for _ in range(NF)]
  cur0 = jnp.int32(-1)
  cur, accs = lax.fori_loop(k_start, k_end, chunk_body, (cur0, accs0))

  flush(cur, accs)

  pltpu.sync_copy(outbuf, out_hbm.at[pl.ds(base_seg, SEG_PER_W)])


@jax.jit
def _segmax(x, batch):
  mesh = plsc.VectorSubcoreMesh(core_axis_name="c", subcore_axis_name="s")
  fn = pl.kernel(
      _body,
      mesh=mesh,
      out_type=jax.ShapeDtypeStruct((NSEG, D), jnp.float32),
      scratch_types=[
          pltpu.VMEM((C, D), jnp.float32),
          pltpu.VMEM((C,), jnp.int32),
          pltpu.VMEM((SEG_PER_W, D), jnp.float32),
          pltpu.VMEM((L,), jnp.int32),
      ],
  )
  return fn(x, batch)


def kernel(x, batch):
  return _segmax(x, batch.astype(jnp.int32))


# speedup vs baseline: 4.8504x; 4.8504x over previous
import jax
import jax.numpy as jnp
from jax import lax
from jax.experimental import pallas as pl
from jax.experimental.pallas import tpu as pltpu
from jax.experimental.pallas import tpu_sc as plsc

N_ROWS = 320000
D = 128
NSEG = 1024
L = 16
NF = D // L
C = 256
NCHUNK = N_ROWS // C
NWORK = 32
SEG_PER_W = NSEG // NWORK
GROUPS = C // L


def _row_vecs(xbuf, row):
  return [xbuf[row, pl.ds(f * L, L)] for f in range(NF)]


def _body(x_hbm, batch_hbm, out_hbm, xbuf, idbuf, outbuf, probe):
  wid = lax.axis_index("s") * 2 + lax.axis_index("c")
  base_seg = wid * SEG_PER_W
  minus_inf = jnp.full((L,), -jnp.inf, jnp.float32)
  iota = lax.iota(jnp.int32, L)

  def init_body(s, carry):
    for f in range(NF):
      outbuf[s, pl.ds(f * L, L)] = minus_inf
    return carry

  lax.fori_loop(0, SEG_PER_W, init_body, 0)

  def count_below(target):
    pos = jnp.int32(0)
    step = 1
    while step * 2 <= NCHUNK:
      step *= 2
    while step >= 1:
      nxt = pos + step
      idx = jnp.minimum(nxt, NCHUNK) - 1
      pltpu.sync_copy(batch_hbm.at[pl.ds(idx * C, L)], probe)
      val = jnp.min(probe[...])
      ok = jnp.logical_and(nxt <= NCHUNK, val < target)
      pos = jnp.where(ok, nxt, pos)
      step //= 2
    return pos

  n_lo = count_below(base_seg)
  k_start = jnp.maximum(n_lo - 1, 0)
  k_end = count_below(base_seg + SEG_PER_W)

  def flush(cur, accs):
    rel = cur - base_seg
    guard = jnp.logical_and(rel >= 0, rel < SEG_PER_W)

    @pl.when(guard)
    def _():
      for f in range(NF):
        outbuf[rel, pl.ds(f * L, L)] = accs[f]

  def group_body(g, carry):
    cur, accs = carry
    ids16 = idbuf[pl.ds(g * L, L)]
    lo = jnp.min(ids16)
    hi = jnp.max(ids16)
    fast = jnp.logical_and(lo == cur, hi == cur)

    def fast_fn(cur, accs):
      for r in range(L):
        row = _row_vecs(xbuf, g * L + r)
        accs = [jnp.maximum(accs[f], row[f]) for f in range(NF)]
      return cur, accs

    def slow_fn(cur, accs):
      for r in range(L):
        id_r = jnp.max(jnp.where(iota == r, ids16, jnp.int32(-2147483647)))
        row = _row_vecs(xbuf, g * L + r)
        changed = id_r != cur

        @pl.when(changed)
        def _():
          flush(cur, accs)

        accs = [
            jnp.where(changed, row[f], jnp.maximum(accs[f], row[f]))
            for f in range(NF)
        ]
        cur = id_r
      return cur, accs

    return lax.cond(fast, fast_fn, slow_fn, cur, accs)

  def chunk_body(k, carry):
    pltpu.sync_copy(x_hbm.at[pl.ds(k * C, C)], xbuf)
    pltpu.sync_copy(batch_hbm.at[pl.ds(k * C, C)], idbuf)
    return lax.fori_loop(0, GROUPS, group_body, carry)

  accs0 = [jnp.full((L,), -jnp.inf, jnp.float32) for _ in range(NF)]
  cur0 = jnp.int32(-1)
  cur, accs = lax.fori_loop(k_start, k_end, chunk_body, (cur0, accs0))

  flush(cur, accs)

  pltpu.sync_copy(outbuf, out_hbm.at[pl.ds(base_seg, SEG_PER_W)])


@jax.jit
def _segmax(x, batch):
  mesh = plsc.VectorSubcoreMesh(core_axis_name="c", subcore_axis_name="s")
  fn = pl.kernel(
      _body,
      mesh=mesh,
      compiler_params=pltpu.CompilerParams(needs_layout_passes=False),
      out_type=jax.ShapeDtypeStruct((NSEG, D), jnp.float32),
      scratch_types=[
          pltpu.VMEM((C, D), jnp.float32),
          pltpu.VMEM((C,), jnp.int32),
          pltpu.VMEM((SEG_PER_W, D), jnp.float32),
          pltpu.VMEM((L,), jnp.int32),
      ],
  )
  return fn(x, batch)


def kernel(x, batch):
  return _segmax(x, batch.astype(jnp.int32))
